# baseline (speedup 1.0000x reference)
import jax
import jax.numpy as jnp
from jax import lax
from jax.experimental import pallas as pl
from jax.experimental.pallas import tpu as pltpu
from jax.experimental.pallas import tpu_sc as plsc

N = 10000
E = 320000
D = 128

NUM_CORES = 2
NUM_SUBCORES = 16
LANES = 16

CHUNK = 80
NROWS_TOTAL = 4096
E_PAD = NROWS_TOTAL * CHUNK
ROWS_PER_SUBCORE = NROWS_TOTAL // NUM_SUBCORES
GROUP = 64
NGROUPS = ROWS_PER_SUBCORE // GROUP

ROWS_MAIN = 624
TAIL_ROWS = N - NUM_SUBCORES * ROWS_MAIN
INIT_ROWS = 16


def _mm_body(x_ref, w0_ref, w1_ref, y0_ref, y1_ref):
    xb = x_ref[...]
    dn = (((1,), (1,)), ((), ()))
    y0_ref[...] = lax.dot_general(xb, w0_ref[...], dn,
                                  preferred_element_type=jnp.float32)
    y1_ref[...] = lax.dot_general(xb, w1_ref[...], dn,
                                  preferred_element_type=jnp.float32)


def _matmuls(x, W0, W1):
    blk = 2000
    grid = N // blk
    return pl.pallas_call(
        _mm_body,
        grid=(grid,),
        in_specs=[
            pl.BlockSpec((blk, D), lambda i: (i, 0)),
            pl.BlockSpec((D, D), lambda i: (0, 0)),
            pl.BlockSpec((D, D), lambda i: (0, 0)),
        ],
        out_specs=[
            pl.BlockSpec((blk, D), lambda i: (i, 0)),
            pl.BlockSpec((blk, D), lambda i: (i, 0)),
        ],
        out_shape=[
            jax.ShapeDtypeStruct((N, D), jnp.float32),
            jax.ShapeDtypeStruct((N, D), jnp.float32),
        ],
    )(x, W0, W1)


def _scale_chunk(rows, vals2, j):
    for t in range(CHUNK // LANES):
        v16 = vals2[j, pl.ds(t * LANES, LANES)]
        for q in range(LANES):
            vv = jnp.full((LANES,), v16[q], jnp.float32)
            i = t * LANES + q
            for k in range(D // LANES):
                sl = pl.ds(k * LANES, LANES)
                rows[i, sl] = rows[i, sl] * vv


def _hop(sid, y_hbm, dst_hbm, src_hbm, vals_hbm, b_hbm, out_hbm,
         dsti, srci, valsv, rows0, rows1, binit, bvec, acc,
         gsem0, gsem1, ssem0, ssem1):
    pltpu.sync_copy(b_hbm, bvec)
    bregs = [bvec[pl.ds(j * LANES, LANES)] for j in range(D // LANES)]

    @pl.loop(0, INIT_ROWS)
    def _(r):
        for j in range(D // LANES):
            binit[r, pl.ds(j * LANES, LANES)] = bregs[j]

    row0 = sid * ROWS_MAIN
    for k in range(ROWS_MAIN // INIT_ROWS):
        pltpu.sync_copy(binit, acc.at[pl.ds(row0 + k * INIT_ROWS, INIT_ROWS)])

    @pl.when(sid == NUM_SUBCORES - 1)
    def _():
        pltpu.sync_copy(binit, acc.at[pl.ds(NUM_SUBCORES * ROWS_MAIN, TAIL_ROWS)])

    plsc.subcore_barrier()

    def wait_gather(sem, buf):
        pltpu.make_async_copy(y_hbm.at[pl.ds(0, CHUNK)], buf, sem).wait()

    def wait_scatter(sem, buf):
        pltpu.make_async_copy(buf, acc.at[pl.ds(0, CHUNK)], sem).wait()

    crow0 = sid * ROWS_PER_SUBCORE

    @pl.loop(0, NGROUPS)
    def _(g):
        grow0 = crow0 + g * GROUP
        pltpu.sync_copy(dst_hbm.at[pl.ds(grow0, GROUP)], dsti)
        pltpu.sync_copy(src_hbm.at[pl.ds(grow0, GROUP)], srci)
        pltpu.sync_copy(vals_hbm.at[pl.ds(grow0, GROUP)], valsv)

        pltpu.async_copy(y_hbm.at[srci.at[0]], rows0, gsem0)
        pltpu.async_copy(y_hbm.at[srci.at[1]], rows1, gsem1)

        @pl.loop(0, GROUP, step=2)
        def _(c):
            wait_gather(gsem0, rows0)
            _scale_chunk(rows0, valsv, c)
            pltpu.async_copy(rows0, acc.at[dsti.at[c]], ssem0, add=True)

            @pl.when(c + 2 < GROUP)
            def _():
                wait_scatter(ssem0, rows0)
                pltpu.async_copy(y_hbm.at[srci.at[c + 2]], rows0, gsem0)

            wait_gather(gsem1, rows1)
            _scale_chunk(rows1, valsv, c + 1)
            pltpu.async_copy(rows1, acc.at[dsti.at[c + 1]], ssem1, add=True)

            @pl.when(c + 3 < GROUP)
            def _():
                wait_scatter(ssem1, rows1)
                pltpu.async_copy(y_hbm.at[srci.at[c + 3]], rows1, gsem1)

        wait_scatter(ssem0, rows0)
        wait_scatter(ssem1, rows1)

    plsc.subcore_barrier()

    pltpu.sync_copy(acc.at[pl.ds(row0, ROWS_MAIN)],
                    out_hbm.at[pl.ds(row0, ROWS_MAIN)])

    @pl.when(sid == NUM_SUBCORES - 1)
    def _():
        tail0 = NUM_SUBCORES * ROWS_MAIN
        pltpu.sync_copy(acc.at[pl.ds(tail0, TAIL_ROWS)],
                        out_hbm.at[pl.ds(tail0, TAIL_ROWS)])


def _spmm_body(y0_hbm, y1_hbm, dst0_hbm, src0_hbm, vals0_hbm,
               dst1_hbm, src1_hbm, vals1_hbm, b0_hbm, b1_hbm,
               f0_hbm, f1_hbm,
               dsti, srci, valsv, rows0, rows1, binit, bvec, acc,
               gsem0, gsem1, ssem0, ssem1):
    cid = lax.axis_index("c")
    sid = lax.axis_index("s")
    scr = (dsti, srci, valsv, rows0, rows1, binit, bvec, acc,
           gsem0, gsem1, ssem0, ssem1)

    @pl.when(cid == 0)
    def _():
        _hop(sid, y0_hbm, dst0_hbm, src0_hbm, vals0_hbm, b0_hbm, f0_hbm, *scr)

    @pl.when(cid == 1)
    def _():
        _hop(sid, y1_hbm, dst1_hbm, src1_hbm, vals1_hbm, b1_hbm, f1_hbm, *scr)


_spmm_sc = pl.kernel(
    _spmm_body,
    out_type=(
        jax.ShapeDtypeStruct((N, D), jnp.float32),
        jax.ShapeDtypeStruct((N, D), jnp.float32),
    ),
    mesh=plsc.VectorSubcoreMesh(
        core_axis_name="c", subcore_axis_name="s",
        num_cores=NUM_CORES, num_subcores=NUM_SUBCORES,
    ),
    scratch_types=[
        pltpu.VMEM((GROUP, CHUNK), jnp.int32),
        pltpu.VMEM((GROUP, CHUNK), jnp.int32),
        pltpu.VMEM((GROUP, CHUNK), jnp.float32),
        pltpu.VMEM((CHUNK, D), jnp.float32),
        pltpu.VMEM((CHUNK, D), jnp.float32),
        pltpu.VMEM((INIT_ROWS, D), jnp.float32),
        pltpu.VMEM((D,), jnp.float32),
        pltpu.VMEM_SHARED((N, D), jnp.float32),
        pltpu.SemaphoreType.DMA,
        pltpu.SemaphoreType.DMA,
        pltpu.SemaphoreType.DMA,
        pltpu.SemaphoreType.DMA,
    ],
)


def _pad_edges(idx, vals):
    pad = E_PAD - E
    dst = jnp.concatenate([idx[0], jnp.zeros((pad,), jnp.int32)])
    src = jnp.concatenate([idx[1], jnp.zeros((pad,), jnp.int32)])
    v = jnp.concatenate([vals, jnp.zeros((pad,), jnp.float32)])
    return (dst.reshape(NROWS_TOTAL, CHUNK), src.reshape(NROWS_TOTAL, CHUNK),
            v.reshape(NROWS_TOTAL, CHUNK))


def kernel(x, edge_index0, edge_vals0, edge_index1, edge_vals1, W0, b0, W1, b1):
    y0, y1 = _matmuls(x, W0, W1)
    dst0, src0, v0 = _pad_edges(edge_index0, edge_vals0)
    dst1, src1, v1 = _pad_edges(edge_index1, edge_vals1)
    f0, f1 = _spmm_sc(y0, y1, dst0, src0, v0, dst1, src1, v1, b0, b1)
    return jnp.concatenate([f0, f1], axis=1)


# speedup vs baseline: 1.0481x; 1.0481x over previous
import jax
import jax.numpy as jnp
from jax import lax
from jax.experimental import pallas as pl
from jax.experimental.pallas import tpu as pltpu
from jax.experimental.pallas import tpu_sc as plsc

N = 10000
E = 320000
D = 128

NUM_CORES = 2
NUM_SUBCORES = 16
LANES = 16

CHUNK = 128
NROWS_TOTAL = 2560
E_PAD = NROWS_TOTAL * CHUNK
ROWS_PER_SUBCORE = NROWS_TOTAL // NUM_SUBCORES
GROUP = 32
NGROUPS = ROWS_PER_SUBCORE // GROUP

ROWS_MAIN = 624
TAIL_ROWS = N - NUM_SUBCORES * ROWS_MAIN
INIT_ROWS = 16


def _mm_body(x_ref, w0_ref, w1_ref, y0_ref, y1_ref):
    xb = x_ref[...]
    dn = (((1,), (1,)), ((), ()))
    y0_ref[...] = lax.dot_general(xb, w0_ref[...], dn,
                                  preferred_element_type=jnp.float32)
    y1_ref[...] = lax.dot_general(xb, w1_ref[...], dn,
                                  preferred_element_type=jnp.float32)


def _matmuls(x, W0, W1):
    blk = 2000
    grid = N // blk
    return pl.pallas_call(
        _mm_body,
        grid=(grid,),
        in_specs=[
            pl.BlockSpec((blk, D), lambda i: (i, 0)),
            pl.BlockSpec((D, D), lambda i: (0, 0)),
            pl.BlockSpec((D, D), lambda i: (0, 0)),
        ],
        out_specs=[
            pl.BlockSpec((blk, D), lambda i: (i, 0)),
            pl.BlockSpec((blk, D), lambda i: (i, 0)),
        ],
        out_shape=[
            jax.ShapeDtypeStruct((N, D), jnp.float32),
            jax.ShapeDtypeStruct((N, D), jnp.float32),
        ],
    )(x, W0, W1)


def _scale_chunk(rows, vals2, j):
    @pl.loop(0, CHUNK // LANES)
    def _(t):
        v16 = vals2[j, pl.ds(t * LANES, LANES)]
        i0 = t * LANES
        for q in range(LANES):
            vv = jnp.full((LANES,), v16[q], jnp.float32)
            for k in range(D // LANES):
                sl = pl.ds(k * LANES, LANES)
                rows[i0 + q, sl] = rows[i0 + q, sl] * vv


def _hop(sid, y_hbm, dst_hbm, src_hbm, vals_hbm, b_hbm, out_hbm,
         dsti, srci, valsv, rows0, rows1, binit, bvec, acc,
         gsem0, gsem1, ssem0, ssem1):
    pltpu.sync_copy(b_hbm, bvec)
    bregs = [bvec[pl.ds(j * LANES, LANES)] for j in range(D // LANES)]

    @pl.loop(0, INIT_ROWS)
    def _(r):
        for j in range(D // LANES):
            binit[r, pl.ds(j * LANES, LANES)] = bregs[j]

    row0 = sid * ROWS_MAIN
    for k in range(ROWS_MAIN // INIT_ROWS):
        pltpu.sync_copy(binit, acc.at[pl.ds(row0 + k * INIT_ROWS, INIT_ROWS)])

    @pl.when(sid == NUM_SUBCORES - 1)
    def _():
        pltpu.sync_copy(binit, acc.at[pl.ds(NUM_SUBCORES * ROWS_MAIN, TAIL_ROWS)])

    plsc.subcore_barrier()

    def wait_gather(sem, buf):
        pltpu.make_async_copy(y_hbm.at[pl.ds(0, CHUNK)], buf, sem).wait()

    def wait_scatter(sem, buf):
        pltpu.make_async_copy(buf, acc.at[pl.ds(0, CHUNK)], sem).wait()

    crow0 = sid * ROWS_PER_SUBCORE

    @pl.loop(0, NGROUPS)
    def _(g):
        grow0 = crow0 + g * GROUP
        pltpu.sync_copy(dst_hbm.at[pl.ds(grow0, GROUP)], dsti)
        pltpu.sync_copy(src_hbm.at[pl.ds(grow0, GROUP)], srci)
        pltpu.sync_copy(vals_hbm.at[pl.ds(grow0, GROUP)], valsv)

        pltpu.async_copy(y_hbm.at[srci.at[0]], rows0, gsem0)
        pltpu.async_copy(y_hbm.at[srci.at[1]], rows1, gsem1)

        @pl.loop(0, GROUP, step=2)
        def _(c):
            wait_gather(gsem0, rows0)
            _scale_chunk(rows0, valsv, c)
            pltpu.async_copy(rows0, acc.at[dsti.at[c]], ssem0, add=True)

            @pl.when(c + 2 < GROUP)
            def _():
                wait_scatter(ssem0, rows0)
                pltpu.async_copy(y_hbm.at[srci.at[c + 2]], rows0, gsem0)

            wait_gather(gsem1, rows1)
            _scale_chunk(rows1, valsv, c + 1)
            pltpu.async_copy(rows1, acc.at[dsti.at[c + 1]], ssem1, add=True)

            @pl.when(c + 3 < GROUP)
            def _():
                wait_scatter(ssem1, rows1)
                pltpu.async_copy(y_hbm.at[srci.at[c + 3]], rows1, gsem1)

        wait_scatter(ssem0, rows0)
        wait_scatter(ssem1, rows1)

    plsc.subcore_barrier()

    pltpu.sync_copy(acc.at[pl.ds(row0, ROWS_MAIN)],
                    out_hbm.at[pl.ds(row0, ROWS_MAIN)])

    @pl.when(sid == NUM_SUBCORES - 1)
    def _():
        tail0 = NUM_SUBCORES * ROWS_MAIN
        pltpu.sync_copy(acc.at[pl.ds(tail0, TAIL_ROWS)],
                        out_hbm.at[pl.ds(tail0, TAIL_ROWS)])


def _spmm_body(y0_hbm, y1_hbm, dst0_hbm, src0_hbm, vals0_hbm,
               dst1_hbm, src1_hbm, vals1_hbm, b0_hbm, b1_hbm,
               f0_hbm, f1_hbm,
               dsti, srci, valsv, rows0, rows1, binit, bvec, acc,
               gsem0, gsem1, ssem0, ssem1):
    cid = lax.axis_index("c")
    sid = lax.axis_index("s")
    scr = (dsti, srci, valsv, rows0, rows1, binit, bvec, acc,
           gsem0, gsem1, ssem0, ssem1)

    @pl.when(cid == 0)
    def _():
        _hop(sid, y0_hbm, dst0_hbm, src0_hbm, vals0_hbm, b0_hbm, f0_hbm, *scr)

    @pl.when(cid == 1)
    def _():
        _hop(sid, y1_hbm, dst1_hbm, src1_hbm, vals1_hbm, b1_hbm, f1_hbm, *scr)


_spmm_sc = pl.kernel(
    _spmm_body,
    out_type=(
        jax.ShapeDtypeStruct((N, D), jnp.float32),
        jax.ShapeDtypeStruct((N, D), jnp.float32),
    ),
    mesh=plsc.VectorSubcoreMesh(
        core_axis_name="c", subcore_axis_name="s",
        num_cores=NUM_CORES, num_subcores=NUM_SUBCORES,
    ),
    scratch_types=[
        pltpu.VMEM((GROUP, CHUNK), jnp.int32),
        pltpu.VMEM((GROUP, CHUNK), jnp.int32),
        pltpu.VMEM((GROUP, CHUNK), jnp.float32),
        pltpu.VMEM((CHUNK, D), jnp.float32),
        pltpu.VMEM((CHUNK, D), jnp.float32),
        pltpu.VMEM((INIT_ROWS, D), jnp.float32),
        pltpu.VMEM((D,), jnp.float32),
        pltpu.VMEM_SHARED((N, D), jnp.float32),
        pltpu.SemaphoreType.DMA,
        pltpu.SemaphoreType.DMA,
        pltpu.SemaphoreType.DMA,
        pltpu.SemaphoreType.DMA,
    ],
)


def _pad_edges(idx, vals):
    pad = E_PAD - E
    dst = jnp.concatenate([idx[0], jnp.zeros((pad,), jnp.int32)])
    src = jnp.concatenate([idx[1], jnp.zeros((pad,), jnp.int32)])
    v = jnp.concatenate([vals, jnp.zeros((pad,), jnp.float32)])
    return (dst.reshape(NROWS_TOTAL, CHUNK), src.reshape(NROWS_TOTAL, CHUNK),
            v.reshape(NROWS_TOTAL, CHUNK))


def kernel(x, edge_index0, edge_vals0, edge_index1, edge_vals1, W0, b0, W1, b1):
    y0, y1 = _matmuls(x, W0, W1)
    dst0, src0, v0 = _pad_edges(edge_index0, edge_vals0)
    dst1, src1, v1 = _pad_edges(edge_index1, edge_vals1)
    f0, f1 = _spmm_sc(y0, y1, dst0, src0, v0, dst1, src1, v1, b0, b1)
    return jnp.concatenate([f0, f1], axis=1)
